# Initial kernel scaffold; baseline (speedup 1.0000x reference)
import jax
import jax.numpy as jnp
from jax.experimental import pallas as pl


def kernel(loc3d, cam_K, cam_W, condition_depth, lf):
    raise NotImplementedError("write your pallas kernel here")


# speedup vs baseline: 15.8460x; 15.8460x over previous
import functools

import numpy as np
import jax
import jax.numpy as jnp
from jax import lax
from jax.experimental import pallas as pl
from jax.experimental.pallas import tpu as pltpu
from jax.experimental.pallas import tpu_sc as plsc

B = 4
C = 256
N = 224
M = 224
NM = N * M
NC = 2
NS = 16
L = 16
CHUNK = NM // 16
VPC = CHUNK // L
LB = B // NC
CPT = C // NS
SUB = CHUNK // 4
MAGIC = np.float32(2.0 ** 23)


def _f32(x):
    return np.float32(x)


def _al(off):
    return pl.multiple_of(off, 8)


def _bfr(x):
    u = plsc.bitcast(x, jnp.int32)
    r = (u + ((lax.shift_right_logical(u, 16) & 1) + 0x7FFF)) & (-65536)
    return plsc.bitcast(r, jnp.float32)


def _body(loc3d, kb_hbm, wb_hbm, cond, lf, out,
          locbuf, kbuf, wbuf, flatbuf, pixbuf, valbuf, outbuf,
          grid1, grid2, sp_flat, sp_vals, sp_pix, sp_pix2):
    cid = lax.axis_index("c")
    sid = lax.axis_index("s")
    lane = lax.iota(jnp.int32, L)

    for lb in range(LB):
        b = cid * LB + lb
        pltpu.sync_copy(kb_hbm.at[pl.ds(_al(b * 192), 192)], kbuf)
        pltpu.sync_copy(wb_hbm.at[pl.ds(_al(b * 192), 192)], wbuf)
        k_ = [_bfr(kbuf[pl.ds(i * L, L)]) for i in range(12)]
        w_ = [_bfr(wbuf[pl.ds(i * L, L)]) for i in range(12)]

        def p1_sub(sub, _):
            for r in range(3):
                pltpu.sync_copy(
                    loc3d.at[pl.ds(
                        _al((b * 3 + r) * NM + sid * CHUNK + sub * SUB), SUB)],
                    locbuf.at[pl.ds(r * SUB, SUB)])

            def p1_body(j, _):
                x = _bfr(locbuf[pl.ds(j * L, L)])
                y = _bfr(locbuf[pl.ds(SUB + j * L, L)])
                z = _bfr(locbuf[pl.ds(2 * SUB + j * L, L)])
                t0 = w_[0] * x + w_[1] * y + w_[2] * z + w_[3]
                t1 = w_[4] * x + w_[5] * y + w_[6] * z + w_[7]
                t2 = w_[8] * x + w_[9] * y + w_[10] * z + w_[11]
                t0 = _bfr(t0)
                t1 = _bfr(t1)
                t2 = _bfr(t2)
                pm0 = k_[0] * t0 + k_[1] * t1 + k_[2] * t2 + k_[3]
                pm1 = k_[4] * t0 + k_[5] * t1 + k_[6] * t2 + k_[7]
                pm2 = k_[8] * t0 + k_[9] * t1 + k_[10] * t2 + k_[11]
                c = pm2 / _f32(112.0)
                p0 = pm0 / c
                p1 = pm1 / c
                r0 = (p0 + MAGIC) - MAGIC
                r1 = (p1 + MAGIC) - MAGIC
                ih = jnp.minimum(jnp.maximum(N - r1.astype(jnp.int32), 0), N - 1)
                iw = jnp.minimum(jnp.maximum(r0.astype(jnp.int32), 0), M - 1)
                flat = ih * M + iw
                ihf = jnp.minimum(jnp.maximum(_f32(N) - r1, _f32(0.0)), _f32(N - 1))
                iwf = jnp.minimum(jnp.maximum(r0, _f32(0.0)), _f32(M - 1))
                pixv = (ihf * _f32(M) + iwf).astype(jnp.int32)
                val = c * _f32(2.0) * _f32(112.0) / _f32(137.0)
                o = sub * SUB + j * L
                flatbuf[pl.ds(o, L)] = flat
                pixbuf[pl.ds(o, L)] = pixv
                valbuf[pl.ds(o, L)] = val
                return _

            lax.fori_loop(0, SUB // L, p1_body, None)
            return _

        lax.fori_loop(0, 4, p1_sub, None)
        off = _al(b * NM + sid * CHUNK)
        pltpu.sync_copy(flatbuf, sp_flat.at[pl.ds(off, CHUNK)])
        pltpu.sync_copy(pixbuf, sp_pix.at[pl.ds(off, CHUNK)])
        pltpu.sync_copy(valbuf, sp_vals.at[pl.ds(off, CHUNK)])

    plsc.subcore_barrier()

    @pl.when(sid < LB)
    def _phase2():
        lb = sid
        b = cid * LB + lb

        def z_body(j, _):
            grid1[pl.ds(j * L, L)] = jnp.zeros((L,), jnp.float32)
            return _
        lax.fori_loop(0, NM // L, z_body, None)

        def sc_chunk(ck, _):
            off = _al(b * NM + ck * CHUNK)
            pltpu.sync_copy(sp_flat.at[pl.ds(off, CHUNK)], flatbuf)
            pltpu.sync_copy(sp_vals.at[pl.ds(off, CHUNK)], valbuf)

            def sc_vec(j, _):
                vidx = flatbuf[pl.ds(j * L, L)]
                vval = valbuf[pl.ds(j * L, L)]
                _cnt, alive = plsc.scan_count(vidx)
                plsc.store_scatter(grid1, [vidx], vval, mask=alive)
                return _

            lax.fori_loop(0, VPC, sc_vec, None)
            return _

        lax.fori_loop(0, 16, sc_chunk, None)

        pltpu.sync_copy(cond.at[pl.ds(_al(b * NM), NM)], grid2)

        def oc_chunk(ck, _):
            off = _al(b * NM + ck * CHUNK)
            pltpu.sync_copy(sp_pix.at[pl.ds(off, CHUNK)], pixbuf)

            def oc_vec(j, _):
                p = pixbuf[pl.ds(j * L, L)]
                cdv = plsc.load_gather(grid1, [p])
                cnv = plsc.load_gather(grid2, [p])
                occ = jnp.logical_and(
                    cdv != _f32(0.0),
                    jnp.abs(cdv - cnv) < _f32(0.02))
                pix2 = jnp.where(occ, p, NM + lane)
                valbuf[pl.ds(j * L, L)] = plsc.bitcast(pix2, jnp.float32)
                return _

            lax.fori_loop(0, VPC, oc_vec, None)
            off2 = _al(lb * NM + ck * CHUNK)
            pltpu.sync_copy(valbuf, sp_pix2.at[pl.ds(off2, CHUNK)])
            return _

        lax.fori_loop(0, 16, oc_chunk, None)

    plsc.subcore_barrier()

    for lb in range(LB):
        b = cid * LB + lb
        pltpu.sync_copy(sp_pix2.at[pl.ds(_al(lb * NM), NM)], grid2)
        grid1[pl.ds(NM, L)] = jnp.zeros((L,), jnp.float32)

        def ch_body(k, _):
            ch = sid * CPT + k
            row_off = _al((b * C + ch) * NM)
            pltpu.sync_copy(lf.at[pl.ds(row_off, NM)], grid1.at[pl.ds(0, NM)])

            def g_chunk(ck, _):
                def g_vec(j, _):
                    idxv = plsc.bitcast(
                        grid2[pl.ds(ck * CHUNK + j * L, L)], jnp.int32)
                    outbuf[pl.ds(j * L, L)] = plsc.load_gather(grid1, [idxv])
                    return _

                lax.fori_loop(0, VPC, g_vec, None)
                pltpu.sync_copy(
                    outbuf, out.at[pl.ds(_al(row_off + ck * CHUNK), CHUNK)])
                return _

            lax.fori_loop(0, 16, g_chunk, None)
            return _

        lax.fori_loop(0, CPT, ch_body, None)


@jax.jit
def _run(loc3d_f, kb, wb, cond_f, lf_f):
    mesh = plsc.VectorSubcoreMesh(core_axis_name="c", subcore_axis_name="s")
    f = pl.kernel(
        _body,
        out_type=jax.ShapeDtypeStruct((B * C * NM,), jnp.float32),
        mesh=mesh,
        compiler_params=pltpu.CompilerParams(needs_layout_passes=False),
        scratch_types=[
            pltpu.VMEM((3 * SUB,), jnp.float32),
            pltpu.VMEM((192,), jnp.float32),
            pltpu.VMEM((192,), jnp.float32),
            pltpu.VMEM((CHUNK,), jnp.int32),
            pltpu.VMEM((CHUNK,), jnp.int32),
            pltpu.VMEM((CHUNK,), jnp.float32),
            pltpu.VMEM((CHUNK,), jnp.float32),
            pltpu.VMEM((NM + L,), jnp.float32),
            pltpu.VMEM((NM,), jnp.float32),
            pltpu.HBM((B * NM,), jnp.int32),
            pltpu.HBM((B * NM,), jnp.float32),
            pltpu.HBM((B * NM,), jnp.int32),
            pltpu.VMEM_SHARED((LB * NM,), jnp.float32),
        ],
    )
    return f(loc3d_f, kb, wb, cond_f, lf_f)


def kernel(loc3d, cam_K, cam_W, condition_depth, lf):
    loc3d_f = loc3d.reshape(B * 3 * NM)
    kb = jnp.broadcast_to(cam_K.reshape(B, 12, 1), (B, 12, L)).reshape(-1)
    wb = jnp.broadcast_to(cam_W.reshape(B, 12, 1), (B, 12, L)).reshape(-1)
    cond_f = condition_depth.reshape(B * NM)
    out = _run(loc3d_f, kb, wb, cond_f, lf.reshape(B * C * NM))
    return out.reshape(B, C, N, M)


# speedup vs baseline: 27.2469x; 1.7195x over previous
import functools

import numpy as np
import jax
import jax.numpy as jnp
from jax import lax
from jax.experimental import pallas as pl
from jax.experimental.pallas import tpu as pltpu
from jax.experimental.pallas import tpu_sc as plsc

B = 4
C = 256
N = 224
M = 224
NM = N * M
NC = 2
NS = 16
L = 16
CHUNK = NM // 16
VPC = CHUNK // L
LB = B // NC
CPT = C // NS
SUB = CHUNK // 4
MAGIC = np.float32(2.0 ** 23)


def _f32(x):
    return np.float32(x)


def _al(off):
    return pl.multiple_of(off, 8)


def _bfr(x):
    u = plsc.bitcast(x, jnp.int32)
    r = (u + ((lax.shift_right_logical(u, 16) & 1) + 0x7FFF)) & (-65536)
    return plsc.bitcast(r, jnp.float32)


def _body(loc3d, kb_hbm, wb_hbm, cond, lf, out,
          locbuf, kbuf, wbuf, flatbuf, pixbuf, valbuf, outbuf,
          grid1, grid2, sp_flat, sp_vals, sp_pix, sp_pix2, osem0, osem1):
    cid = lax.axis_index("c")
    sid = lax.axis_index("s")
    lane = lax.iota(jnp.int32, L)

    for lb in range(LB):
        b = cid * LB + lb
        pltpu.sync_copy(kb_hbm.at[pl.ds(_al(b * 192), 192)], kbuf)
        pltpu.sync_copy(wb_hbm.at[pl.ds(_al(b * 192), 192)], wbuf)
        k_ = [_bfr(kbuf[pl.ds(i * L, L)]) for i in range(12)]
        w_ = [_bfr(wbuf[pl.ds(i * L, L)]) for i in range(12)]

        def p1_sub(sub, _):
            for r in range(3):
                pltpu.sync_copy(
                    loc3d.at[pl.ds(
                        _al((b * 3 + r) * NM + sid * CHUNK + sub * SUB), SUB)],
                    locbuf.at[pl.ds(r * SUB, SUB)])

            def p1_body(j, _):
                x = _bfr(locbuf[pl.ds(j * L, L)])
                y = _bfr(locbuf[pl.ds(SUB + j * L, L)])
                z = _bfr(locbuf[pl.ds(2 * SUB + j * L, L)])
                t0 = w_[0] * x + w_[1] * y + w_[2] * z + w_[3]
                t1 = w_[4] * x + w_[5] * y + w_[6] * z + w_[7]
                t2 = w_[8] * x + w_[9] * y + w_[10] * z + w_[11]
                t0 = _bfr(t0)
                t1 = _bfr(t1)
                t2 = _bfr(t2)
                pm0 = k_[0] * t0 + k_[1] * t1 + k_[2] * t2 + k_[3]
                pm1 = k_[4] * t0 + k_[5] * t1 + k_[6] * t2 + k_[7]
                pm2 = k_[8] * t0 + k_[9] * t1 + k_[10] * t2 + k_[11]
                c = pm2 / _f32(112.0)
                p0 = pm0 / c
                p1 = pm1 / c
                r0 = (p0 + MAGIC) - MAGIC
                r1 = (p1 + MAGIC) - MAGIC
                ih = jnp.minimum(jnp.maximum(N - r1.astype(jnp.int32), 0), N - 1)
                iw = jnp.minimum(jnp.maximum(r0.astype(jnp.int32), 0), M - 1)
                flat = ih * M + iw
                ihf = jnp.minimum(jnp.maximum(_f32(N) - r1, _f32(0.0)), _f32(N - 1))
                iwf = jnp.minimum(jnp.maximum(r0, _f32(0.0)), _f32(M - 1))
                pixv = (ihf * _f32(M) + iwf).astype(jnp.int32)
                val = c * _f32(2.0) * _f32(112.0) / _f32(137.0)
                o = sub * SUB + j * L
                flatbuf[pl.ds(o, L)] = flat
                pixbuf[pl.ds(o, L)] = pixv
                valbuf[pl.ds(o, L)] = val
                return _

            plsc.parallel_loop(0, SUB // L, unroll=4)(
                lambda j: p1_body(j, None))
            return _

        lax.fori_loop(0, 4, p1_sub, None)
        off = _al(b * NM + sid * CHUNK)
        pltpu.sync_copy(flatbuf, sp_flat.at[pl.ds(off, CHUNK)])
        pltpu.sync_copy(pixbuf, sp_pix.at[pl.ds(off, CHUNK)])
        pltpu.sync_copy(valbuf, sp_vals.at[pl.ds(off, CHUNK)])

    plsc.subcore_barrier()

    @pl.when(sid < LB)
    def _phase2():
        lb = sid
        b = cid * LB + lb

        def z_body(j, _):
            grid1[pl.ds(j * L, L)] = jnp.zeros((L,), jnp.float32)
            return _
        lax.fori_loop(0, NM // L, z_body, None)

        def sc_chunk(ck, _):
            off = _al(b * NM + ck * CHUNK)
            pltpu.sync_copy(sp_flat.at[pl.ds(off, CHUNK)], flatbuf)
            pltpu.sync_copy(sp_vals.at[pl.ds(off, CHUNK)], valbuf)

            def sc_vec(j, _):
                vidx = flatbuf[pl.ds(j * L, L)]
                vval = valbuf[pl.ds(j * L, L)]
                _cnt, alive = plsc.scan_count(vidx)
                plsc.store_scatter(grid1, [vidx], vval, mask=alive)
                return _

            lax.fori_loop(0, VPC, sc_vec, None)
            return _

        lax.fori_loop(0, 16, sc_chunk, None)

        pltpu.sync_copy(cond.at[pl.ds(_al(b * NM), NM)], grid2)

        def oc_chunk(ck, _):
            off = _al(b * NM + ck * CHUNK)
            pltpu.sync_copy(sp_pix.at[pl.ds(off, CHUNK)], pixbuf)

            def oc_vec(j, _):
                p = pixbuf[pl.ds(j * L, L)]
                cdv = plsc.load_gather(grid1, [p])
                cnv = plsc.load_gather(grid2, [p])
                occ = jnp.logical_and(
                    cdv != _f32(0.0),
                    jnp.abs(cdv - cnv) < _f32(0.02))
                pix2 = jnp.where(occ, p, NM + lane)
                valbuf[pl.ds(j * L, L)] = plsc.bitcast(pix2, jnp.float32)
                return _

            plsc.parallel_loop(0, VPC, unroll=4)(
                lambda j: oc_vec(j, None))
            off2 = _al(lb * NM + ck * CHUNK)
            pltpu.sync_copy(valbuf, sp_pix2.at[pl.ds(off2, CHUNK)])
            return _

        lax.fori_loop(0, 16, oc_chunk, None)

    plsc.subcore_barrier()

    for lb in range(LB):
        b = cid * LB + lb
        pltpu.sync_copy(sp_pix2.at[pl.ds(_al(lb * NM), NM)], grid2)
        grid1[pl.ds(NM, L)] = jnp.zeros((L,), jnp.float32)

        def ch_body(k, _):
            ch = sid * CPT + k
            row_off = _al((b * C + ch) * NM)
            pltpu.sync_copy(lf.at[pl.ds(row_off, NM)], grid1.at[pl.ds(0, NM)])

            for ck in range(16):
                par = (ck % 2) * CHUNK
                sem = osem0 if ck % 2 == 0 else osem1
                obuf = outbuf.at[pl.ds(par, CHUNK)]
                if ck >= 2:
                    pltpu.make_async_copy(
                        obuf, out.at[pl.ds(_al(row_off), CHUNK)], sem).wait()

                @plsc.parallel_loop(0, VPC, unroll=8)
                def g_vec(j, _ck=ck, _par=par):
                    idxv = plsc.bitcast(
                        grid2[pl.ds(_ck * CHUNK + j * L, L)], jnp.int32)
                    outbuf[pl.ds(_par + j * L, L)] = plsc.load_gather(
                        grid1, [idxv])

                pltpu.async_copy(
                    obuf, out.at[pl.ds(_al(row_off + ck * CHUNK), CHUNK)], sem)
            for p2 in range(2):
                pltpu.make_async_copy(
                    outbuf.at[pl.ds(p2 * CHUNK, CHUNK)],
                    out.at[pl.ds(_al(row_off), CHUNK)],
                    osem0 if p2 == 0 else osem1).wait()
            return _

        lax.fori_loop(0, CPT, ch_body, None)


@jax.jit
def _run(loc3d_f, kb, wb, cond_f, lf_f):
    mesh = plsc.VectorSubcoreMesh(core_axis_name="c", subcore_axis_name="s")
    f = pl.kernel(
        _body,
        out_type=jax.ShapeDtypeStruct((B * C * NM,), jnp.float32),
        mesh=mesh,
        compiler_params=pltpu.CompilerParams(needs_layout_passes=False),
        scratch_types=[
            pltpu.VMEM((3 * SUB,), jnp.float32),
            pltpu.VMEM((192,), jnp.float32),
            pltpu.VMEM((192,), jnp.float32),
            pltpu.VMEM((CHUNK,), jnp.int32),
            pltpu.VMEM((CHUNK,), jnp.int32),
            pltpu.VMEM((CHUNK,), jnp.float32),
            pltpu.VMEM((2 * CHUNK,), jnp.float32),
            pltpu.VMEM((NM + L,), jnp.float32),
            pltpu.VMEM((NM,), jnp.float32),
            pltpu.HBM((B * NM,), jnp.int32),
            pltpu.HBM((B * NM,), jnp.float32),
            pltpu.HBM((B * NM,), jnp.int32),
            pltpu.VMEM_SHARED((LB * NM,), jnp.float32),
            pltpu.SemaphoreType.DMA,
            pltpu.SemaphoreType.DMA,
        ],
    )
    return f(loc3d_f, kb, wb, cond_f, lf_f)


def kernel(loc3d, cam_K, cam_W, condition_depth, lf):
    loc3d_f = loc3d.reshape(B * 3 * NM)
    kb = jnp.broadcast_to(cam_K.reshape(B, 12, 1), (B, 12, L)).reshape(-1)
    wb = jnp.broadcast_to(cam_W.reshape(B, 12, 1), (B, 12, L)).reshape(-1)
    cond_f = condition_depth.reshape(B * NM)
    out = _run(loc3d_f, kb, wb, cond_f, lf.reshape(B * C * NM))
    return out.reshape(B, C, N, M)


# speedup vs baseline: 27.4972x; 1.0092x over previous
import numpy as np
import jax
import jax.numpy as jnp
from jax import lax
from jax.experimental import pallas as pl
from jax.experimental.pallas import tpu as pltpu
from jax.experimental.pallas import tpu_sc as plsc

B = 4
C = 256
N = 224
M = 224
NM = N * M
NC = 2
NS = 16
L = 16
CHUNK = NM // 16
VPC = CHUNK // L
LB = B // NC
CPT = C // NS
SUB = CHUNK // 4
MAGIC = np.float32(2.0 ** 23)
NRING = 4


def _f32(x):
    return np.float32(x)


def _al(off):
    return pl.multiple_of(off, 8)


def _bfr(x):
    u = plsc.bitcast(x, jnp.int32)
    r = (u + ((lax.shift_right_logical(u, 16) & 1) + 0x7FFF)) & (-65536)
    return plsc.bitcast(r, jnp.float32)


def _body(loc3d, kb_hbm, wb_hbm, cond, lf, out,
          locbuf, kbuf, wbuf, ws, grid1, grid2,
          sp_flat, sp_vals, sp_pix, sp_pix2, sems):
    cid = lax.axis_index("c")
    sid = lax.axis_index("s")
    lane = lax.iota(jnp.int32, L)

    FLAT0, PIX0, VAL0 = 0, CHUNK, 2 * CHUNK

    for lb in range(LB):
        b = cid * LB + lb
        pltpu.sync_copy(kb_hbm.at[pl.ds(_al(b * 192), 192)], kbuf)
        pltpu.sync_copy(wb_hbm.at[pl.ds(_al(b * 192), 192)], wbuf)
        k_ = [_bfr(kbuf[pl.ds(i * L, L)]) for i in range(12)]
        w_ = [_bfr(wbuf[pl.ds(i * L, L)]) for i in range(12)]

        def p1_sub(sub, _):
            for r in range(3):
                pltpu.sync_copy(
                    loc3d.at[pl.ds(
                        _al((b * 3 + r) * NM + sid * CHUNK + sub * SUB), SUB)],
                    locbuf.at[pl.ds(r * SUB, SUB)])

            @plsc.parallel_loop(0, SUB // L, unroll=4)
            def p1_body(j):
                x = _bfr(locbuf[pl.ds(j * L, L)])
                y = _bfr(locbuf[pl.ds(SUB + j * L, L)])
                z = _bfr(locbuf[pl.ds(2 * SUB + j * L, L)])
                t0 = w_[0] * x + w_[1] * y + w_[2] * z + w_[3]
                t1 = w_[4] * x + w_[5] * y + w_[6] * z + w_[7]
                t2 = w_[8] * x + w_[9] * y + w_[10] * z + w_[11]
                t0 = _bfr(t0)
                t1 = _bfr(t1)
                t2 = _bfr(t2)
                pm0 = k_[0] * t0 + k_[1] * t1 + k_[2] * t2 + k_[3]
                pm1 = k_[4] * t0 + k_[5] * t1 + k_[6] * t2 + k_[7]
                pm2 = k_[8] * t0 + k_[9] * t1 + k_[10] * t2 + k_[11]
                c = pm2 / _f32(112.0)
                p0 = pm0 / c
                p1 = pm1 / c
                r0 = (p0 + MAGIC) - MAGIC
                r1 = (p1 + MAGIC) - MAGIC
                ih = jnp.minimum(
                    jnp.maximum(N - r1.astype(jnp.int32), 0), N - 1)
                iw = jnp.minimum(jnp.maximum(r0.astype(jnp.int32), 0), M - 1)
                flat = ih * M + iw
                ihf = jnp.minimum(
                    jnp.maximum(_f32(N) - r1, _f32(0.0)), _f32(N - 1))
                iwf = jnp.minimum(jnp.maximum(r0, _f32(0.0)), _f32(M - 1))
                pixv = (ihf * _f32(M) + iwf).astype(jnp.int32)
                val = c * _f32(2.0) * _f32(112.0) / _f32(137.0)
                o = sub * SUB + j * L
                ws[pl.ds(FLAT0 + o, L)] = plsc.bitcast(flat, jnp.float32)
                ws[pl.ds(PIX0 + o, L)] = plsc.bitcast(pixv, jnp.float32)
                ws[pl.ds(VAL0 + o, L)] = val

            return _

        lax.fori_loop(0, 4, p1_sub, None)
        off = _al(b * NM + sid * CHUNK)
        pltpu.sync_copy(ws.at[pl.ds(FLAT0, CHUNK)],
                        sp_flat.at[pl.ds(off, CHUNK)])
        pltpu.sync_copy(ws.at[pl.ds(PIX0, CHUNK)],
                        sp_pix.at[pl.ds(off, CHUNK)])
        pltpu.sync_copy(ws.at[pl.ds(VAL0, CHUNK)],
                        sp_vals.at[pl.ds(off, CHUNK)])

    plsc.subcore_barrier()

    @pl.when(sid < LB)
    def _phase2():
        lb = sid
        b = cid * LB + lb

        def z_body(j, _):
            grid1[pl.ds(j * L, L)] = jnp.zeros((L,), jnp.float32)
            return _
        lax.fori_loop(0, NM // L, z_body, None)

        def sc_chunk(ck, _):
            off = _al(b * NM + ck * CHUNK)
            pltpu.sync_copy(sp_flat.at[pl.ds(off, CHUNK)],
                            ws.at[pl.ds(FLAT0, CHUNK)])
            pltpu.sync_copy(sp_vals.at[pl.ds(off, CHUNK)],
                            ws.at[pl.ds(VAL0, CHUNK)])

            def sc_vec(j, _):
                vidx = plsc.bitcast(ws[pl.ds(FLAT0 + j * L, L)], jnp.int32)
                vval = ws[pl.ds(VAL0 + j * L, L)]
                _cnt, alive = plsc.scan_count(vidx)
                plsc.store_scatter(grid1, [vidx], vval, mask=alive)
                return _

            lax.fori_loop(0, VPC, sc_vec, None)
            return _

        lax.fori_loop(0, 16, sc_chunk, None)

        pltpu.sync_copy(cond.at[pl.ds(_al(b * NM), NM)], grid2)

        def oc_chunk(ck, _):
            off = _al(b * NM + ck * CHUNK)
            pltpu.sync_copy(sp_pix.at[pl.ds(off, CHUNK)],
                            ws.at[pl.ds(PIX0, CHUNK)])

            @plsc.parallel_loop(0, VPC, unroll=4)
            def oc_vec(j):
                p = plsc.bitcast(ws[pl.ds(PIX0 + j * L, L)], jnp.int32)
                cdv = plsc.load_gather(grid1, [p])
                cnv = plsc.load_gather(grid2, [p])
                occ = jnp.logical_and(
                    cdv != _f32(0.0),
                    jnp.abs(cdv - cnv) < _f32(0.02))
                pix2 = jnp.where(occ, p, NM + lane)
                ws[pl.ds(VAL0 + j * L, L)] = plsc.bitcast(pix2, jnp.float32)

            pltpu.sync_copy(ws.at[pl.ds(VAL0, CHUNK)],
                            sp_pix2.at[pl.ds(off, CHUNK)])
            return _

        lax.fori_loop(0, 16, oc_chunk, None)

    plsc.subcore_barrier()

    for lb in range(LB):
        b = cid * LB + lb
        pltpu.sync_copy(sp_pix2.at[pl.ds(_al(b * NM), NM)], grid2)
        grid1[pl.ds(NM, L)] = jnp.zeros((L,), jnp.float32)

        def ch_body(k, _):
            ch = sid * CPT + k
            row_off = _al((b * C + ch) * NM)
            pltpu.sync_copy(lf.at[pl.ds(row_off, NM)], grid1.at[pl.ds(0, NM)])

            for ck in range(16):
                slot = ck % NRING
                par = slot * CHUNK
                sem = sems.at[slot]
                obuf = ws.at[pl.ds(par, CHUNK)]
                if ck >= NRING:
                    pltpu.make_async_copy(
                        obuf, out.at[pl.ds(_al(row_off), CHUNK)], sem).wait()

                @plsc.parallel_loop(0, VPC, unroll=8)
                def g_vec(j, _ck=ck, _par=par):
                    idxv = plsc.bitcast(
                        grid2[pl.ds(_ck * CHUNK + j * L, L)], jnp.int32)
                    ws[pl.ds(_par + j * L, L)] = plsc.load_gather(
                        grid1, [idxv])

                pltpu.async_copy(
                    obuf, out.at[pl.ds(_al(row_off + ck * CHUNK), CHUNK)], sem)
            for slot in range(NRING):
                pltpu.make_async_copy(
                    ws.at[pl.ds(slot * CHUNK, CHUNK)],
                    out.at[pl.ds(_al(row_off), CHUNK)],
                    sems.at[slot]).wait()
            return _

        lax.fori_loop(0, CPT, ch_body, None)


@jax.jit
def _run(loc3d_f, kb, wb, cond_f, lf_f):
    mesh = plsc.VectorSubcoreMesh(core_axis_name="c", subcore_axis_name="s")
    f = pl.kernel(
        _body,
        out_type=jax.ShapeDtypeStruct((B * C * NM,), jnp.float32),
        mesh=mesh,
        compiler_params=pltpu.CompilerParams(needs_layout_passes=False),
        scratch_types=[
            pltpu.VMEM((3 * SUB,), jnp.float32),
            pltpu.VMEM((192,), jnp.float32),
            pltpu.VMEM((192,), jnp.float32),
            pltpu.VMEM((NRING * CHUNK,), jnp.float32),
            pltpu.VMEM((NM + L,), jnp.float32),
            pltpu.VMEM((NM,), jnp.float32),
            pltpu.HBM((B * NM,), jnp.float32),
            pltpu.HBM((B * NM,), jnp.float32),
            pltpu.HBM((B * NM,), jnp.float32),
            pltpu.HBM((B * NM,), jnp.float32),
            pltpu.SemaphoreType.DMA((NRING,)),
        ],
    )
    return f(loc3d_f, kb, wb, cond_f, lf_f)


def kernel(loc3d, cam_K, cam_W, condition_depth, lf):
    loc3d_f = loc3d.reshape(B * 3 * NM)
    kb = jnp.broadcast_to(cam_K.reshape(B, 12, 1), (B, 12, L)).reshape(-1)
    wb = jnp.broadcast_to(cam_W.reshape(B, 12, 1), (B, 12, L)).reshape(-1)
    cond_f = condition_depth.reshape(B * NM)
    out = _run(loc3d_f, kb, wb, cond_f, lf.reshape(B * C * NM))
    return out.reshape(B, C, N, M)


# speedup vs baseline: 27.8366x; 1.0123x over previous
import numpy as np
import jax
import jax.numpy as jnp
from jax import lax
from jax.experimental import pallas as pl
from jax.experimental.pallas import tpu as pltpu
from jax.experimental.pallas import tpu_sc as plsc

B = 4
C = 256
N = 224
M = 224
NM = N * M
NC = 2
NS = 16
L = 16
CHUNK = NM // 16
VPC = CHUNK // L
LB = B // NC
CPT = C // NS
SUB = CHUNK // 4
MAGIC = np.float32(2.0 ** 23)
NRING = 4
OC = 2 * CHUNK
NOC = NM // OC
VPO = OC // L


def _f32(x):
    return np.float32(x)


def _al(off):
    return pl.multiple_of(off, 8)


def _bfr(x):
    u = plsc.bitcast(x, jnp.int32)
    r = (u + ((lax.shift_right_logical(u, 16) & 1) + 0x7FFF)) & (-65536)
    return plsc.bitcast(r, jnp.float32)


def _body(loc3d, kb_hbm, wb_hbm, cond, lf, out,
          locbuf, kbuf, wbuf, ws, grid1, grid2,
          sp_flat, sp_vals, sp_pix, sp_pix2, sems):
    cid = lax.axis_index("c")
    sid = lax.axis_index("s")
    lane = lax.iota(jnp.int32, L)

    FLAT0, PIX0, VAL0 = 0, CHUNK, 2 * CHUNK

    for lb in range(LB):
        b = cid * LB + lb
        pltpu.sync_copy(kb_hbm.at[pl.ds(_al(b * 192), 192)], kbuf)
        pltpu.sync_copy(wb_hbm.at[pl.ds(_al(b * 192), 192)], wbuf)
        k_ = [_bfr(kbuf[pl.ds(i * L, L)]) for i in range(12)]
        w_ = [_bfr(wbuf[pl.ds(i * L, L)]) for i in range(12)]

        def p1_sub(sub, _):
            for r in range(3):
                pltpu.sync_copy(
                    loc3d.at[pl.ds(
                        _al((b * 3 + r) * NM + sid * CHUNK + sub * SUB), SUB)],
                    locbuf.at[pl.ds(r * SUB, SUB)])

            @plsc.parallel_loop(0, SUB // L, unroll=4)
            def p1_body(j):
                x = _bfr(locbuf[pl.ds(j * L, L)])
                y = _bfr(locbuf[pl.ds(SUB + j * L, L)])
                z = _bfr(locbuf[pl.ds(2 * SUB + j * L, L)])
                t0 = w_[0] * x + w_[1] * y + w_[2] * z + w_[3]
                t1 = w_[4] * x + w_[5] * y + w_[6] * z + w_[7]
                t2 = w_[8] * x + w_[9] * y + w_[10] * z + w_[11]
                t0 = _bfr(t0)
                t1 = _bfr(t1)
                t2 = _bfr(t2)
                pm0 = k_[0] * t0 + k_[1] * t1 + k_[2] * t2 + k_[3]
                pm1 = k_[4] * t0 + k_[5] * t1 + k_[6] * t2 + k_[7]
                pm2 = k_[8] * t0 + k_[9] * t1 + k_[10] * t2 + k_[11]
                c = pm2 / _f32(112.0)
                p0 = pm0 / c
                p1 = pm1 / c
                r0 = (p0 + MAGIC) - MAGIC
                r1 = (p1 + MAGIC) - MAGIC
                ih = jnp.minimum(
                    jnp.maximum(N - r1.astype(jnp.int32), 0), N - 1)
                iw = jnp.minimum(jnp.maximum(r0.astype(jnp.int32), 0), M - 1)
                flat = ih * M + iw
                ihf = jnp.minimum(
                    jnp.maximum(_f32(N) - r1, _f32(0.0)), _f32(N - 1))
                iwf = jnp.minimum(jnp.maximum(r0, _f32(0.0)), _f32(M - 1))
                pixv = (ihf * _f32(M) + iwf).astype(jnp.int32)
                val = c * _f32(2.0) * _f32(112.0) / _f32(137.0)
                o = sub * SUB + j * L
                ws[pl.ds(FLAT0 + o, L)] = plsc.bitcast(flat, jnp.float32)
                ws[pl.ds(PIX0 + o, L)] = plsc.bitcast(pixv, jnp.float32)
                ws[pl.ds(VAL0 + o, L)] = val

            return _

        lax.fori_loop(0, 4, p1_sub, None)
        off = _al(b * NM + sid * CHUNK)
        pltpu.sync_copy(ws.at[pl.ds(FLAT0, CHUNK)],
                        sp_flat.at[pl.ds(off, CHUNK)])
        pltpu.sync_copy(ws.at[pl.ds(PIX0, CHUNK)],
                        sp_pix.at[pl.ds(off, CHUNK)])
        pltpu.sync_copy(ws.at[pl.ds(VAL0, CHUNK)],
                        sp_vals.at[pl.ds(off, CHUNK)])

    plsc.subcore_barrier()

    @pl.when(sid < LB)
    def _phase2():
        lb = sid
        b = cid * LB + lb

        def z_body(j, _):
            grid1[pl.ds(j * L, L)] = jnp.zeros((L,), jnp.float32)
            return _
        lax.fori_loop(0, NM // L, z_body, None)

        def sc_chunk(ck, _):
            off = _al(b * NM + ck * CHUNK)
            pltpu.sync_copy(sp_flat.at[pl.ds(off, CHUNK)],
                            ws.at[pl.ds(FLAT0, CHUNK)])
            pltpu.sync_copy(sp_vals.at[pl.ds(off, CHUNK)],
                            ws.at[pl.ds(VAL0, CHUNK)])

            def sc_vec(j, _):
                vidx = plsc.bitcast(ws[pl.ds(FLAT0 + j * L, L)], jnp.int32)
                vval = ws[pl.ds(VAL0 + j * L, L)]
                _cnt, alive = plsc.scan_count(vidx)
                plsc.store_scatter(grid1, [vidx], vval, mask=alive)
                return _

            lax.fori_loop(0, VPC, sc_vec, None)
            return _

        lax.fori_loop(0, 16, sc_chunk, None)

        pltpu.sync_copy(cond.at[pl.ds(_al(b * NM), NM)], grid2)

        def oc_chunk(ck, _):
            off = _al(b * NM + ck * CHUNK)
            pltpu.sync_copy(sp_pix.at[pl.ds(off, CHUNK)],
                            ws.at[pl.ds(PIX0, CHUNK)])

            @plsc.parallel_loop(0, VPC, unroll=4)
            def oc_vec(j):
                p = plsc.bitcast(ws[pl.ds(PIX0 + j * L, L)], jnp.int32)
                cdv = plsc.load_gather(grid1, [p])
                cnv = plsc.load_gather(grid2, [p])
                occ = jnp.logical_and(
                    cdv != _f32(0.0),
                    jnp.abs(cdv - cnv) < _f32(0.02))
                pix2 = jnp.where(occ, p, NM + lane)
                ws[pl.ds(VAL0 + j * L, L)] = plsc.bitcast(pix2, jnp.float32)

            pltpu.sync_copy(ws.at[pl.ds(VAL0, CHUNK)],
                            sp_pix2.at[pl.ds(off, CHUNK)])
            return _

        lax.fori_loop(0, 16, oc_chunk, None)

    plsc.subcore_barrier()

    for lb in range(LB):
        b = cid * LB + lb
        pltpu.sync_copy(sp_pix2.at[pl.ds(_al(b * NM), NM)], grid2)
        grid1[pl.ds(NM, L)] = jnp.zeros((L,), jnp.float32)

        def ch_body(k, _):
            ch = sid * CPT + k
            row_off = _al((b * C + ch) * NM)
            pltpu.sync_copy(lf.at[pl.ds(row_off, NM)], grid1.at[pl.ds(0, NM)])

            for ck in range(NOC):
                slot = ck % NRING
                par = slot * OC
                sem = sems.at[slot]
                obuf = ws.at[pl.ds(par, OC)]
                if ck >= NRING:
                    pltpu.make_async_copy(
                        obuf, out.at[pl.ds(_al(row_off), OC)], sem).wait()
                else:
                    @pl.when(k > 0)
                    def _wait_prev(_obuf=obuf, _sem=sem):
                        pltpu.make_async_copy(
                            _obuf, out.at[pl.ds(_al(row_off), OC)],
                            _sem).wait()

                @plsc.parallel_loop(0, VPO, unroll=8)
                def g_vec(j, _ck=ck, _par=par):
                    idxv = plsc.bitcast(
                        grid2[pl.ds(_ck * OC + j * L, L)], jnp.int32)
                    ws[pl.ds(_par + j * L, L)] = plsc.load_gather(
                        grid1, [idxv])

                pltpu.async_copy(
                    obuf, out.at[pl.ds(_al(row_off + ck * OC), OC)], sem)
            return _

        lax.fori_loop(0, CPT, ch_body, None)
        for slot in range(NRING):
            pltpu.make_async_copy(
                ws.at[pl.ds(slot * OC, OC)],
                out.at[pl.ds(_al(b * C * NM), OC)],
                sems.at[slot]).wait()


@jax.jit
def _run(loc3d_f, kb, wb, cond_f, lf_f):
    mesh = plsc.VectorSubcoreMesh(core_axis_name="c", subcore_axis_name="s")
    f = pl.kernel(
        _body,
        out_type=jax.ShapeDtypeStruct((B * C * NM,), jnp.float32),
        mesh=mesh,
        compiler_params=pltpu.CompilerParams(needs_layout_passes=False),
        scratch_types=[
            pltpu.VMEM((3 * SUB,), jnp.float32),
            pltpu.VMEM((192,), jnp.float32),
            pltpu.VMEM((192,), jnp.float32),
            pltpu.VMEM((NRING * OC,), jnp.float32),
            pltpu.VMEM((NM + L,), jnp.float32),
            pltpu.VMEM((NM,), jnp.float32),
            pltpu.HBM((B * NM,), jnp.float32),
            pltpu.HBM((B * NM,), jnp.float32),
            pltpu.HBM((B * NM,), jnp.float32),
            pltpu.HBM((B * NM,), jnp.float32),
            pltpu.SemaphoreType.DMA((NRING,)),
        ],
    )
    return f(loc3d_f, kb, wb, cond_f, lf_f)


def kernel(loc3d, cam_K, cam_W, condition_depth, lf):
    loc3d_f = loc3d.reshape(B * 3 * NM)
    kb = jnp.broadcast_to(cam_K.reshape(B, 12, 1), (B, 12, L)).reshape(-1)
    wb = jnp.broadcast_to(cam_W.reshape(B, 12, 1), (B, 12, L)).reshape(-1)
    cond_f = condition_depth.reshape(B * NM)
    out = _run(loc3d_f, kb, wb, cond_f, lf.reshape(B * C * NM))
    return out.reshape(B, C, N, M)


# speedup vs baseline: 27.8380x; 1.0001x over previous
import numpy as np
import jax
import jax.numpy as jnp
from jax import lax
from jax.experimental import pallas as pl
from jax.experimental.pallas import tpu as pltpu
from jax.experimental.pallas import tpu_sc as plsc

B = 4
C = 256
N = 224
M = 224
NM = N * M
NC = 2
NS = 16
L = 16
CHUNK = NM // 16
VPC = CHUNK // L
LB = B // NC
CPT = C // NS
SUB = CHUNK // 4
MAGIC = np.float32(2.0 ** 23)
NRING = 4
OC = 2 * CHUNK
NOC = NM // OC
VPO = OC // L


def _f32(x):
    return np.float32(x)


def _al(off):
    return pl.multiple_of(off, 8)


def _bfr(x):
    u = plsc.bitcast(x, jnp.int32)
    r = (u + ((lax.shift_right_logical(u, 16) & 1) + 0x7FFF)) & (-65536)
    return plsc.bitcast(r, jnp.float32)


def _body(loc3d, kb_hbm, wb_hbm, cond, lf, out,
          locbuf, kbuf, wbuf, ws, grid1, grid2,
          sp_flat, sp_vals, sp_pix, sp_pix2, sems, isems):
    cid = lax.axis_index("c")
    sid = lax.axis_index("s")
    lane = lax.iota(jnp.int32, L)

    FLAT0, PIX0, VAL0 = 0, CHUNK, 2 * CHUNK

    for lb in range(LB):
        b = cid * LB + lb
        pltpu.sync_copy(kb_hbm.at[pl.ds(_al(b * 192), 192)], kbuf)
        pltpu.sync_copy(wb_hbm.at[pl.ds(_al(b * 192), 192)], wbuf)
        k_ = [_bfr(kbuf[pl.ds(i * L, L)]) for i in range(12)]
        w_ = [_bfr(wbuf[pl.ds(i * L, L)]) for i in range(12)]

        def p1_sub(sub, _):
            for r in range(3):
                pltpu.sync_copy(
                    loc3d.at[pl.ds(
                        _al((b * 3 + r) * NM + sid * CHUNK + sub * SUB), SUB)],
                    locbuf.at[pl.ds(r * SUB, SUB)])

            @plsc.parallel_loop(0, SUB // L, unroll=4)
            def p1_body(j):
                x = _bfr(locbuf[pl.ds(j * L, L)])
                y = _bfr(locbuf[pl.ds(SUB + j * L, L)])
                z = _bfr(locbuf[pl.ds(2 * SUB + j * L, L)])
                t0 = w_[0] * x + w_[1] * y + w_[2] * z + w_[3]
                t1 = w_[4] * x + w_[5] * y + w_[6] * z + w_[7]
                t2 = w_[8] * x + w_[9] * y + w_[10] * z + w_[11]
                t0 = _bfr(t0)
                t1 = _bfr(t1)
                t2 = _bfr(t2)
                pm0 = k_[0] * t0 + k_[1] * t1 + k_[2] * t2 + k_[3]
                pm1 = k_[4] * t0 + k_[5] * t1 + k_[6] * t2 + k_[7]
                pm2 = k_[8] * t0 + k_[9] * t1 + k_[10] * t2 + k_[11]
                c = pm2 / _f32(112.0)
                p0 = pm0 / c
                p1 = pm1 / c
                r0 = (p0 + MAGIC) - MAGIC
                r1 = (p1 + MAGIC) - MAGIC
                ih = jnp.minimum(
                    jnp.maximum(N - r1.astype(jnp.int32), 0), N - 1)
                iw = jnp.minimum(jnp.maximum(r0.astype(jnp.int32), 0), M - 1)
                flat = ih * M + iw
                ihf = jnp.minimum(
                    jnp.maximum(_f32(N) - r1, _f32(0.0)), _f32(N - 1))
                iwf = jnp.minimum(jnp.maximum(r0, _f32(0.0)), _f32(M - 1))
                pixv = (ihf * _f32(M) + iwf).astype(jnp.int32)
                val = c * _f32(2.0) * _f32(112.0) / _f32(137.0)
                o = sub * SUB + j * L
                ws[pl.ds(FLAT0 + o, L)] = plsc.bitcast(flat, jnp.float32)
                ws[pl.ds(PIX0 + o, L)] = plsc.bitcast(pixv, jnp.float32)
                ws[pl.ds(VAL0 + o, L)] = val

            return _

        lax.fori_loop(0, 4, p1_sub, None)
        off = _al(b * NM + sid * CHUNK)
        pltpu.sync_copy(ws.at[pl.ds(FLAT0, CHUNK)],
                        sp_flat.at[pl.ds(off, CHUNK)])
        pltpu.sync_copy(ws.at[pl.ds(PIX0, CHUNK)],
                        sp_pix.at[pl.ds(off, CHUNK)])
        pltpu.sync_copy(ws.at[pl.ds(VAL0, CHUNK)],
                        sp_vals.at[pl.ds(off, CHUNK)])

    plsc.subcore_barrier()

    @pl.when(sid < LB)
    def _phase2():
        lb = sid
        b = cid * LB + lb

        def z_body(j, _):
            grid1[pl.ds(j * L, L)] = jnp.zeros((L,), jnp.float32)
            return _
        lax.fori_loop(0, NM // L, z_body, None)

        def sc_chunk(ck, _):
            off = _al(b * NM + ck * CHUNK)
            pltpu.sync_copy(sp_flat.at[pl.ds(off, CHUNK)],
                            ws.at[pl.ds(FLAT0, CHUNK)])
            pltpu.sync_copy(sp_vals.at[pl.ds(off, CHUNK)],
                            ws.at[pl.ds(VAL0, CHUNK)])

            def sc_vec(j, _):
                vidx = plsc.bitcast(ws[pl.ds(FLAT0 + j * L, L)], jnp.int32)
                vval = ws[pl.ds(VAL0 + j * L, L)]
                _cnt, alive = plsc.scan_count(vidx)
                plsc.store_scatter(grid1, [vidx], vval, mask=alive)
                return _

            lax.fori_loop(0, VPC, sc_vec, None)
            return _

        lax.fori_loop(0, 16, sc_chunk, None)

        pltpu.sync_copy(cond.at[pl.ds(_al(b * NM), NM)], grid2)

        def oc_chunk(ck, _):
            off = _al(b * NM + ck * CHUNK)
            pltpu.sync_copy(sp_pix.at[pl.ds(off, CHUNK)],
                            ws.at[pl.ds(PIX0, CHUNK)])

            @plsc.parallel_loop(0, VPC, unroll=4)
            def oc_vec(j):
                p = plsc.bitcast(ws[pl.ds(PIX0 + j * L, L)], jnp.int32)
                cdv = plsc.load_gather(grid1, [p])
                cnv = plsc.load_gather(grid2, [p])
                occ = jnp.logical_and(
                    cdv != _f32(0.0),
                    jnp.abs(cdv - cnv) < _f32(0.02))
                pix2 = jnp.where(occ, p, NM + lane)
                ws[pl.ds(VAL0 + j * L, L)] = plsc.bitcast(pix2, jnp.float32)

            pltpu.sync_copy(ws.at[pl.ds(VAL0, CHUNK)],
                            sp_pix2.at[pl.ds(off, CHUNK)])
            return _

        lax.fori_loop(0, 16, oc_chunk, None)

    plsc.subcore_barrier()

    for lb in range(LB):
        b = cid * LB + lb
        pltpu.sync_copy(sp_pix2.at[pl.ds(_al(b * NM), NM)], grid2)
        grid1[pl.ds(NM, L)] = jnp.zeros((L,), jnp.float32)

        def ch_body(k, _):
            ch = sid * CPT + k
            row_off = _al((b * C + ch) * NM)
            Q = NM // 4
            for q in range(4):
                pltpu.async_copy(
                    lf.at[pl.ds(_al(row_off + q * Q), Q)],
                    grid1.at[pl.ds(q * Q, Q)], isems.at[q])
            for q in range(4):
                pltpu.make_async_copy(
                    lf.at[pl.ds(_al(row_off), Q)],
                    grid1.at[pl.ds(0, Q)], isems.at[q]).wait()

            for ck in range(NOC):
                slot = ck % NRING
                par = slot * OC
                sem = sems.at[slot]
                obuf = ws.at[pl.ds(par, OC)]
                if ck >= NRING:
                    pltpu.make_async_copy(
                        obuf, out.at[pl.ds(_al(row_off), OC)], sem).wait()
                else:
                    @pl.when(k > 0)
                    def _wait_prev(_obuf=obuf, _sem=sem):
                        pltpu.make_async_copy(
                            _obuf, out.at[pl.ds(_al(row_off), OC)],
                            _sem).wait()

                @plsc.parallel_loop(0, VPO, unroll=8)
                def g_vec(j, _ck=ck, _par=par):
                    idxv = plsc.bitcast(
                        grid2[pl.ds(_ck * OC + j * L, L)], jnp.int32)
                    ws[pl.ds(_par + j * L, L)] = plsc.load_gather(
                        grid1, [idxv])

                pltpu.async_copy(
                    obuf, out.at[pl.ds(_al(row_off + ck * OC), OC)], sem)
            return _

        lax.fori_loop(0, CPT, ch_body, None)
        for slot in range(NRING):
            pltpu.make_async_copy(
                ws.at[pl.ds(slot * OC, OC)],
                out.at[pl.ds(_al(b * C * NM), OC)],
                sems.at[slot]).wait()


@jax.jit
def _run(loc3d_f, kb, wb, cond_f, lf_f):
    mesh = plsc.VectorSubcoreMesh(core_axis_name="c", subcore_axis_name="s")
    f = pl.kernel(
        _body,
        out_type=jax.ShapeDtypeStruct((B * C * NM,), jnp.float32),
        mesh=mesh,
        compiler_params=pltpu.CompilerParams(needs_layout_passes=False),
        scratch_types=[
            pltpu.VMEM((3 * SUB,), jnp.float32),
            pltpu.VMEM((192,), jnp.float32),
            pltpu.VMEM((192,), jnp.float32),
            pltpu.VMEM((NRING * OC,), jnp.float32),
            pltpu.VMEM((NM + L,), jnp.float32),
            pltpu.VMEM((NM,), jnp.float32),
            pltpu.HBM((B * NM,), jnp.float32),
            pltpu.HBM((B * NM,), jnp.float32),
            pltpu.HBM((B * NM,), jnp.float32),
            pltpu.HBM((B * NM,), jnp.float32),
            pltpu.SemaphoreType.DMA((NRING,)),
            pltpu.SemaphoreType.DMA((4,)),
        ],
    )
    return f(loc3d_f, kb, wb, cond_f, lf_f)


def kernel(loc3d, cam_K, cam_W, condition_depth, lf):
    loc3d_f = loc3d.reshape(B * 3 * NM)
    kb = jnp.broadcast_to(cam_K.reshape(B, 12, 1), (B, 12, L)).reshape(-1)
    wb = jnp.broadcast_to(cam_W.reshape(B, 12, 1), (B, 12, L)).reshape(-1)
    cond_f = condition_depth.reshape(B * NM)
    out = _run(loc3d_f, kb, wb, cond_f, lf.reshape(B * C * NM))
    return out.reshape(B, C, N, M)
